# Initial kernel scaffold; baseline (speedup 1.0000x reference)
import functools

import jax
import jax.numpy as jnp
from jax import lax
from jax.experimental import pallas as pl
from jax.experimental.pallas import tpu as pltpu
from jax.experimental.pallas import tpu_sc as plsc

N, E, D = 10000, 320000, 128
ROWS_BLK = 1000
N_BLOCKS = N // ROWS_BLK
NTILES = 32
EDGES_PER_TILE = E // NTILES
CHUNK = 80
NCHUNK = EDGES_PER_TILE // CHUNK
ROWS_PER_SUBCORE = N // 16


def _leaky(x):
    return jnp.where(x > 0, x, 0.2 * x)


def _pre_body(x_ref, w1t_ref, b1_ref, w2t_ref, b2_ref, a1_ref, a2_ref,
              ab_ref, xj_ref, ai_ref, aj_ref):
    x = x_ref[...]
    xi = _leaky(jnp.dot(x, w1t_ref[...], preferred_element_type=jnp.float32)
                + b1_ref[...])
    xj = _leaky(jnp.dot(x, w2t_ref[...], preferred_element_type=jnp.float32)
                + b2_ref[...])
    xj_ref[...] = xj
    ai_ref[...] = jnp.sum(xi * a1_ref[...], axis=1, keepdims=True) + ab_ref[0, 0]
    aj_ref[...] = jnp.sum(xj * a2_ref[...], axis=1, keepdims=True) + ab_ref[0, 1]


def _run_pre(x0, w1t, b1r, w2t, b2r, a1r, a2r, abr):
    full = lambda: pl.BlockSpec((1, D), lambda i: (0, 0))
    return pl.pallas_call(
        _pre_body,
        grid=(N_BLOCKS,),
        in_specs=[
            pl.BlockSpec((ROWS_BLK, D), lambda i: (i, 0)),
            pl.BlockSpec((D, D), lambda i: (0, 0)),
            full(),
            pl.BlockSpec((D, D), lambda i: (0, 0)),
            full(), full(), full(),
        ],
        out_specs=[
            pl.BlockSpec((ROWS_BLK, D), lambda i: (i, 0)),
            pl.BlockSpec((ROWS_BLK, 1), lambda i: (i, 0)),
            pl.BlockSpec((ROWS_BLK, 1), lambda i: (i, 0)),
        ],
        out_shape=[
            jax.ShapeDtypeStruct((N, D), jnp.float32),
            jax.ShapeDtypeStruct((N, 1), jnp.float32),
            jax.ShapeDtypeStruct((N, 1), jnp.float32),
        ],
    )(x0, w1t, b1r, w2t, b2r, a1r, a2r, abr)


@functools.partial(
    pl.kernel,
    mesh=plsc.VectorSubcoreMesh(core_axis_name="c", subcore_axis_name="s"),
    out_type=jax.ShapeDtypeStruct((2 * N, D), jnp.float32),
    scratch_types=[
        pltpu.VMEM((CHUNK,), jnp.int32),
        pltpu.VMEM((CHUNK,), jnp.int32),
        pltpu.VMEM((N,), jnp.float32),
        pltpu.VMEM((N,), jnp.float32),
        pltpu.VMEM((CHUNK,), jnp.float32),
        pltpu.VMEM((CHUNK, D), jnp.float32),
        pltpu.VMEM_SHARED((N, D), jnp.float32),
        pltpu.SemaphoreType.DMA,
    ],
)
def _sc_edge_kernel(row_hbm, col_hbm, xj_hbm, ai_hbm, aj_hbm, zeros_hbm,
                    out_hbm, row_v, col_v, ai_l, aj_l, att_v, rows_v,
                    agg_sh, sem):
    c = lax.axis_index("c")
    s = lax.axis_index("s")
    wid = c * 16 + s
    rslice = pl.ds(s * ROWS_PER_SUBCORE, ROWS_PER_SUBCORE)
    pltpu.sync_copy(zeros_hbm, agg_sh.at[rslice])
    pltpu.sync_copy(ai_hbm, ai_l)
    pltpu.sync_copy(aj_hbm, aj_l)
    plsc.subcore_barrier()

    base = wid * EDGES_PER_TILE

    def chunk_body(k, carry):
        off = base + k * CHUNK
        pltpu.sync_copy(row_hbm.at[pl.ds(off, CHUNK)], row_v)
        pltpu.sync_copy(col_hbm.at[pl.ds(off, CHUNK)], col_v)
        cp = pltpu.async_copy(xj_hbm.at[col_v], rows_v, sem)

        def att_body(g, carry2):
            r16 = row_v[pl.ds(g * 16, 16)]
            c16 = col_v[pl.ds(g * 16, 16)]
            ar = plsc.load_gather(ai_l, [r16])
            ac = plsc.load_gather(aj_l, [c16])
            att_v[pl.ds(g * 16, 16)] = 1.0 / (1.0 + jnp.exp(-(ar + ac)))
            return carry2

        lax.fori_loop(0, CHUNK // 16, att_body, 0)
        cp.wait()

        def scale_body(e, carry2):
            a16 = plsc.load_gather(att_v, [jnp.full((16,), e, jnp.int32)])
            for j in range(D // 16):
                sl = pl.ds(j * 16, 16)
                rows_v[e, sl] = rows_v[e, sl] * a16
            return carry2

        lax.fori_loop(0, CHUNK, scale_body, 0)

        pltpu.sync_copy(rows_v, agg_sh.at[row_v], add=True)
        return carry

    lax.fori_loop(0, NCHUNK, chunk_body, 0)
    plsc.subcore_barrier()
    pltpu.sync_copy(agg_sh.at[rslice],
                    out_hbm.at[pl.ds(c * N + s * ROWS_PER_SUBCORE,
                                     ROWS_PER_SUBCORE)])


def _post_body(x_ref, w1t_ref, b1_ref, p_ref, wot_ref, bo_ref, g_ref, be_ref,
               y_ref):
    x = x_ref[...]
    xi = _leaky(jnp.dot(x, w1t_ref[...], preferred_element_type=jnp.float32)
                + b1_ref[...])
    agg = p_ref[0] + p_ref[1]
    agg = 1.0 / (1.0 + jnp.exp(-agg))
    out = (jnp.dot(agg, wot_ref[...], preferred_element_type=jnp.float32)
           + bo_ref[...] + xi + x)
    mean = jnp.mean(out, axis=-1, keepdims=True)
    ctr = out - mean
    var = jnp.mean(ctr * ctr, axis=-1, keepdims=True)
    y_ref[...] = ctr * lax.rsqrt(var + 1e-5) * g_ref[...] + be_ref[...]


def _run_post(x0, w1t, b1r, partials, wot, bor, g1r, be1r):
    full = lambda: pl.BlockSpec((1, D), lambda i: (0, 0))
    return pl.pallas_call(
        _post_body,
        grid=(N_BLOCKS,),
        in_specs=[
            pl.BlockSpec((ROWS_BLK, D), lambda i: (i, 0)),
            pl.BlockSpec((D, D), lambda i: (0, 0)),
            full(),
            pl.BlockSpec((2, ROWS_BLK, D), lambda i: (0, i, 0)),
            pl.BlockSpec((D, D), lambda i: (0, 0)),
            full(), full(), full(),
        ],
        out_specs=pl.BlockSpec((ROWS_BLK, D), lambda i: (i, 0)),
        out_shape=jax.ShapeDtypeStruct((N, D), jnp.float32),
    )(x0, w1t, b1r, partials, wot, bor, g1r, be1r)


def kernel(x0, x1, edge_index, W1, b1, W2, b2, a1w, a1b, a2w, a2b, Wo, bo,
           g1, be1):
    row = edge_index[0]
    col = edge_index[1]
    b1r = b1.reshape(1, D)
    b2r = b2.reshape(1, D)
    abr = jnp.concatenate([a1b, a2b, jnp.zeros((D - 2,), jnp.float32)])
    abr = abr.reshape(1, D)

    xj0, ai0, aj0 = _run_pre(x0, W1.T, b1r, W2.T, b2r, a1w, a2w, abr)

    zeros = jnp.zeros((ROWS_PER_SUBCORE, D), jnp.float32)
    partials = _sc_edge_kernel(row, col, xj0, ai0.reshape(N), aj0.reshape(N),
                               zeros)
    partials = partials.reshape(2, N, D)

    return _run_post(x0, W1.T, b1r, partials, Wo.T, bo.reshape(1, D),
                     g1.reshape(1, D), be1.reshape(1, D))


# speedup vs baseline: 12.0651x; 12.0651x over previous
import functools

import jax
import jax.numpy as jnp
from jax import lax
from jax.experimental import pallas as pl
from jax.experimental.pallas import tpu as pltpu
from jax.experimental.pallas import tpu_sc as plsc

N, E, D = 10000, 320000, 128
ROWS_BLK = 1000
N_BLOCKS = N // ROWS_BLK
NTILES = 32
EDGES_PER_TILE = E // NTILES
CHUNK = 80
NCHUNK = EDGES_PER_TILE // CHUNK
ROWS_PER_SUBCORE = 640
NP = 16 * ROWS_PER_SUBCORE


def _leaky(x):
    return jnp.where(x > 0, x, 0.2 * x)


def _pre_body(x_ref, w1t_ref, b1_ref, w2t_ref, b2_ref, a1_ref, a2_ref,
              ab_ref, xj_ref, ai_ref, aj_ref):
    x = x_ref[...]
    xi = _leaky(jnp.dot(x, w1t_ref[...], preferred_element_type=jnp.float32)
                + b1_ref[...])
    xj = _leaky(jnp.dot(x, w2t_ref[...], preferred_element_type=jnp.float32)
                + b2_ref[...])
    xj_ref[...] = xj
    ai_ref[...] = jnp.sum(xi * a1_ref[...], axis=1, keepdims=True) + ab_ref[0, 0]
    aj_ref[...] = jnp.sum(xj * a2_ref[...], axis=1, keepdims=True) + ab_ref[0, 1]


def _run_pre(x0, w1t, b1r, w2t, b2r, a1r, a2r, abr):
    full = lambda: pl.BlockSpec((1, D), lambda i: (0, 0))
    return pl.pallas_call(
        _pre_body,
        grid=(N_BLOCKS,),
        in_specs=[
            pl.BlockSpec((ROWS_BLK, D), lambda i: (i, 0)),
            pl.BlockSpec((D, D), lambda i: (0, 0)),
            full(),
            pl.BlockSpec((D, D), lambda i: (0, 0)),
            full(), full(), full(), full(),
        ],
        out_specs=[
            pl.BlockSpec((ROWS_BLK, D), lambda i: (i, 0)),
            pl.BlockSpec((ROWS_BLK, 1), lambda i: (i, 0)),
            pl.BlockSpec((ROWS_BLK, 1), lambda i: (i, 0)),
        ],
        out_shape=[
            jax.ShapeDtypeStruct((N, D), jnp.float32),
            jax.ShapeDtypeStruct((N, 1), jnp.float32),
            jax.ShapeDtypeStruct((N, 1), jnp.float32),
        ],
    )(x0, w1t, b1r, w2t, b2r, a1r, a2r, abr)


@functools.partial(
    pl.kernel,
    mesh=plsc.VectorSubcoreMesh(core_axis_name="c", subcore_axis_name="s"),
    out_type=jax.ShapeDtypeStruct((2 * NP, D), jnp.float32),
    compiler_params=pltpu.CompilerParams(needs_layout_passes=False),
    scratch_types=[
        pltpu.VMEM((CHUNK,), jnp.int32),
        pltpu.VMEM((CHUNK,), jnp.int32),
        pltpu.VMEM((N,), jnp.float32),
        pltpu.VMEM((N,), jnp.float32),
        pltpu.VMEM((CHUNK,), jnp.float32),
        pltpu.VMEM((CHUNK, D), jnp.float32),
        pltpu.VMEM_SHARED((NP, D), jnp.float32),
        pltpu.SemaphoreType.DMA,
    ],
)
def _sc_edge_kernel(row_hbm, col_hbm, xj_hbm, ai_hbm, aj_hbm, zeros_hbm,
                    out_hbm, row_v, col_v, ai_l, aj_l, att_v, rows_v,
                    agg_sh, sem):
    c = lax.axis_index("c")
    s = lax.axis_index("s")
    wid = c * 16 + s
    rslice = pl.ds(s * ROWS_PER_SUBCORE, ROWS_PER_SUBCORE)
    pltpu.sync_copy(zeros_hbm, agg_sh.at[rslice])
    pltpu.sync_copy(ai_hbm, ai_l)
    pltpu.sync_copy(aj_hbm, aj_l)
    plsc.subcore_barrier()

    base = wid * EDGES_PER_TILE

    def chunk_body(k, carry):
        off = base + k * CHUNK
        pltpu.sync_copy(row_hbm.at[pl.ds(off, CHUNK)], row_v)
        pltpu.sync_copy(col_hbm.at[pl.ds(off, CHUNK)], col_v)
        cp = pltpu.async_copy(xj_hbm.at[col_v], rows_v, sem)

        def att_body(g, carry2):
            r16 = row_v[pl.ds(g * 16, 16)]
            c16 = col_v[pl.ds(g * 16, 16)]
            ar = plsc.load_gather(ai_l, [r16])
            ac = plsc.load_gather(aj_l, [c16])
            att_v[pl.ds(g * 16, 16)] = 1.0 / (1.0 + jnp.exp(-(ar + ac)))
            return carry2

        lax.fori_loop(0, CHUNK // 16, att_body, 0)
        cp.wait()

        def scale_body(e, carry2):
            a16 = plsc.load_gather(att_v, [jnp.full((16,), e, jnp.int32)])
            for j in range(D // 16):
                sl = pl.ds(j * 16, 16)
                rows_v[e, sl] = rows_v[e, sl] * a16
            return carry2

        lax.fori_loop(0, CHUNK, scale_body, 0)

        pltpu.sync_copy(rows_v, agg_sh.at[row_v], add=True)
        return carry

    lax.fori_loop(0, NCHUNK, chunk_body, 0)
    plsc.subcore_barrier()
    pltpu.sync_copy(agg_sh.at[rslice],
                    out_hbm.at[pl.ds(c * NP + s * ROWS_PER_SUBCORE,
                                     ROWS_PER_SUBCORE)])


def _post_body(x_ref, w1t_ref, b1_ref, p_ref, wot_ref, bo_ref, g_ref, be_ref,
               y_ref):
    x = x_ref[...]
    xi = _leaky(jnp.dot(x, w1t_ref[...], preferred_element_type=jnp.float32)
                + b1_ref[...])
    agg = p_ref[0] + p_ref[1]
    agg = 1.0 / (1.0 + jnp.exp(-agg))
    out = (jnp.dot(agg, wot_ref[...], preferred_element_type=jnp.float32)
           + bo_ref[...] + xi + x)
    mean = jnp.mean(out, axis=-1, keepdims=True)
    ctr = out - mean
    var = jnp.mean(ctr * ctr, axis=-1, keepdims=True)
    y_ref[...] = ctr * lax.rsqrt(var + 1e-5) * g_ref[...] + be_ref[...]


def _run_post(x0, w1t, b1r, partials, wot, bor, g1r, be1r):
    full = lambda: pl.BlockSpec((1, D), lambda i: (0, 0))
    return pl.pallas_call(
        _post_body,
        grid=(N_BLOCKS,),
        in_specs=[
            pl.BlockSpec((ROWS_BLK, D), lambda i: (i, 0)),
            pl.BlockSpec((D, D), lambda i: (0, 0)),
            full(),
            pl.BlockSpec((2, ROWS_BLK, D), lambda i: (0, i, 0)),
            pl.BlockSpec((D, D), lambda i: (0, 0)),
            full(), full(), full(),
        ],
        out_specs=pl.BlockSpec((ROWS_BLK, D), lambda i: (i, 0)),
        out_shape=jax.ShapeDtypeStruct((N, D), jnp.float32),
    )(x0, w1t, b1r, partials, wot, bor, g1r, be1r)


def kernel(x0, x1, edge_index, W1, b1, W2, b2, a1w, a1b, a2w, a2b, Wo, bo,
           g1, be1):
    row = edge_index[0]
    col = edge_index[1]
    b1r = b1.reshape(1, D)
    b2r = b2.reshape(1, D)
    abr = jnp.concatenate([a1b, a2b, jnp.zeros((D - 2,), jnp.float32)])
    abr = abr.reshape(1, D)

    xj0, ai0, aj0 = _run_pre(x0, W1.T, b1r, W2.T, b2r, a1w, a2w, abr)

    zeros = jnp.zeros((ROWS_PER_SUBCORE, D), jnp.float32)
    partials = _sc_edge_kernel(row, col, xj0, ai0.reshape(N), aj0.reshape(N),
                               zeros)
    partials = partials.reshape(2, NP, D)[:, :N]

    return _run_post(x0, W1.T, b1r, partials, Wo.T, bo.reshape(1, D),
                     g1.reshape(1, D), be1.reshape(1, D))
